# Initial kernel scaffold; baseline (speedup 1.0000x reference)
import jax
import jax.numpy as jnp
from jax.experimental import pallas as pl


def kernel(feat, indexes, features, labels):
    raise NotImplementedError("write your pallas kernel here")


# speedup vs baseline: 5.8715x; 5.8715x over previous
import functools

import jax
import jax.numpy as jnp
from jax import lax
from jax.experimental import pallas as pl
from jax.experimental.pallas import tpu as pltpu
from jax.experimental.pallas import tpu_sc as plsc

B = 1024
F = 64
M = 100000
L = 10000
TEMP = 0.05

NC = 2
NS = 16
NW = NC * NS
RPW = M // NW
CHUNK = 125
NCH = RPW // CHUNK
LPW = L // NS
TPW = B // NW
CW = 8

@functools.cache
def _build_sc_segment_sum():
    mesh = plsc.VectorSubcoreMesh(core_axis_name="c", subcore_axis_name="s")

    @functools.partial(
        pl.kernel,
        mesh=mesh,
        compiler_params=pltpu.CompilerParams(use_tc_tiling_on_sc=False),
        out_type=[
            jax.ShapeDtypeStruct((NC * L, F), jnp.float32),
            jax.ShapeDtypeStruct((NC * L, CW), jnp.float32),
            jax.ShapeDtypeStruct((B,), jnp.int32),
        ],
        scratch_types=[
            pltpu.VMEM((CHUNK, F), jnp.float32),
            pltpu.VMEM((NCH, CHUNK), jnp.int32),
            pltpu.VMEM((CHUNK, CW), jnp.float32),
            pltpu.VMEM((TPW,), jnp.int32),
            pltpu.VMEM((TPW,), jnp.int32),
            pltpu.VMEM_SHARED((L, F), jnp.float32),
            pltpu.VMEM_SHARED((L, CW), jnp.float32),
            pltpu.SemaphoreType.DMA,
        ],
    )
    def sc_segment_sum(
        features_hbm, labels2d_hbm, labels1d_hbm, indexes_hbm, ones_hbm,
        zf_hbm, zc_hbm,
        out_feat, out_cnt, out_tgt,
        rows_v, lab_v, ones_v, idx_v, tgt_v, acc_f, acc_c, sem,
    ):
        cid = lax.axis_index("c")
        sid = lax.axis_index("s")
        wid = sid * NC + cid

        pltpu.sync_copy(zf_hbm.at[pl.ds(sid * LPW, LPW)], acc_f.at[pl.ds(sid * LPW, LPW)])
        pltpu.sync_copy(zc_hbm.at[pl.ds(sid * LPW, LPW)], acc_c.at[pl.ds(sid * LPW, LPW)])

        pltpu.sync_copy(labels2d_hbm.at[pl.ds(wid * NCH, NCH)], lab_v)
        pltpu.sync_copy(ones_hbm, ones_v)

        pltpu.sync_copy(indexes_hbm.at[pl.ds(wid * TPW, TPW)], idx_v)
        pltpu.async_copy(labels1d_hbm.at[idx_v], tgt_v, sem).wait()
        pltpu.sync_copy(tgt_v, out_tgt.at[pl.ds(wid * TPW, TPW)])

        plsc.subcore_barrier()

        base = wid * RPW
        for j in range(NCH):
            pltpu.sync_copy(features_hbm.at[pl.ds(base + j * CHUNK, CHUNK)], rows_v)
            pltpu.sync_copy(rows_v, acc_f.at[lab_v.at[j]], add=True)
            pltpu.sync_copy(ones_v, acc_c.at[lab_v.at[j]], add=True)

        plsc.subcore_barrier()

        pltpu.sync_copy(
            acc_f.at[pl.ds(sid * LPW, LPW)],
            out_feat.at[pl.ds(cid * L + sid * LPW, LPW)],
        )
        pltpu.sync_copy(
            acc_c.at[pl.ds(sid * LPW, LPW)],
            out_cnt.at[pl.ds(cid * L + sid * LPW, LPW)],
        )

    return sc_segment_sum


TL = 1000


def _tc_body(cf2_ref, cnt2_ref, feat_ref, tgt_ref, out_ref, denom_ref, tnum_ref):
    i = pl.program_id(0)

    @pl.when(i == 0)
    def _init():
        denom_ref[...] = jnp.zeros_like(denom_ref)
        tnum_ref[...] = jnp.zeros_like(tnum_ref)

    x = feat_ref[...]
    nrm = jnp.sqrt(jnp.sum(x * x, axis=1, keepdims=True)) + 1e-12
    xn = x / nrm

    cf = cf2_ref[0] + cf2_ref[1]
    cnt = cnt2_ref[0, :, 0:1] + cnt2_ref[1, :, 0:1]
    sim = lax.dot_general(
        cf, xn, (((1,), (1,)), ((), ())),
        preferred_element_type=jnp.float32,
        precision=lax.Precision.HIGHEST,
    )
    mask = cnt > 0.0
    sim = sim / (TEMP * jnp.where(mask, cnt, 1.0))
    e = jnp.where(mask, jnp.exp(sim), 0.0)
    denom_ref[...] += jnp.sum(e, axis=0, keepdims=True)

    row = i * TL + lax.broadcasted_iota(jnp.int32, (TL, 1), 0)
    tmatch = row == tgt_ref[...]
    tnum_ref[...] += jnp.sum(jnp.where(tmatch, sim, 0.0), axis=0, keepdims=True)

    @pl.when(i == pl.num_programs(0) - 1)
    def _fin():
        p = jnp.exp(tnum_ref[...]) / (denom_ref[...] + 1e-6)
        lp = jnp.log(p + 1e-6)
        out_ref[0, 0] = -jnp.mean(lp)


def _tc_loss(cf2, cnt2, feat, tgt2d):
    return pl.pallas_call(
        _tc_body,
        grid=(L // TL,),
        in_specs=[
            pl.BlockSpec((NC, TL, F), lambda i: (0, i, 0)),
            pl.BlockSpec((NC, TL, CW), lambda i: (0, i, 0)),
            pl.BlockSpec((B, F), lambda i: (0, 0)),
            pl.BlockSpec((1, B), lambda i: (0, 0)),
        ],
        out_specs=pl.BlockSpec((1, 1), lambda i: (0, 0), memory_space=pltpu.SMEM),
        out_shape=jax.ShapeDtypeStruct((1, 1), jnp.float32),
        scratch_shapes=[
            pltpu.VMEM((1, B), jnp.float32),
            pltpu.VMEM((1, B), jnp.float32),
        ],
    )(cf2, cnt2, feat, tgt2d)


def kernel(feat, indexes, features, labels):
    labels2d = labels.reshape(NW * NCH, CHUNK)
    ones = jnp.ones((CHUNK, CW), jnp.float32)
    zf = jnp.zeros((L, F), jnp.float32)
    zc = jnp.zeros((L, CW), jnp.float32)

    cf, cnt, targets = _build_sc_segment_sum()(
        features, labels2d, labels, indexes, ones, zf, zc
    )
    loss = _tc_loss(
        cf.reshape(NC, L, F),
        cnt.reshape(NC, L, CW),
        feat,
        targets.reshape(1, B),
    )
    return loss[0, 0]
